# baseline (speedup 1.0000x reference)
import jax
import jax.numpy as jnp
from jax.experimental import pallas as pl
from jax.experimental.pallas import tpu as pltpu


def kernel(x, row_w, col_w):
    n, dim, h, w = x.shape
    half = dim // 2
    hw = h * w

    def body(row_ref, col_ref, out_ref, buf, sem):
        lane = jax.lax.broadcasted_iota(jnp.int32, (w, hw), 1)
        src = jax.lax.broadcasted_iota(jnp.int32, (w, hw), 0)
        p = (lane % w == src).astype(jnp.float32)
        lane_h = jax.lax.broadcasted_iota(jnp.int32, (h, hw), 1)
        src_h = jax.lax.broadcasted_iota(jnp.int32, (h, hw), 0)
        q = (lane_h // w == src_h).astype(jnp.float32)
        xe = jax.lax.dot_general(
            col_ref[...], p, (((0,), (0,)), ((), ())),
            preferred_element_type=jnp.float32,
        )
        ye = jax.lax.dot_general(
            row_ref[...], q, (((0,), (0,)), ((), ())),
            preferred_element_type=jnp.float32,
        )
        buf[0:half, :] = xe
        buf[half:dim, :] = ye
        cp = pltpu.make_async_copy(buf, out_ref.at[0], sem)
        cp.start()
        cp.wait()
        filled = 1
        while filled < n:
            m = min(filled, n - filled)
            cp = pltpu.make_async_copy(
                out_ref.at[pl.ds(0, m)], out_ref.at[pl.ds(filled, m)], sem
            )
            cp.start()
            cp.wait()
            filled += m

    out = pl.pallas_call(
        body,
        in_specs=[
            pl.BlockSpec(memory_space=pltpu.VMEM),
            pl.BlockSpec(memory_space=pltpu.VMEM),
        ],
        out_specs=pl.BlockSpec(memory_space=pl.ANY),
        out_shape=jax.ShapeDtypeStruct((n, dim, hw), jnp.float32),
        scratch_shapes=[
            pltpu.VMEM((dim, hw), jnp.float32),
            pltpu.SemaphoreType.DMA,
        ],
    )(row_w[:h], col_w[:w])
    return out.reshape(n, dim, h, w)


# speedup vs baseline: 14.6320x; 14.6320x over previous
import jax
import jax.numpy as jnp
from jax.experimental import pallas as pl
from jax.experimental.pallas import tpu as pltpu

_CHUNKS_PER_SLOT = 4


def kernel(x, row_w, col_w):
    n, dim, h, w = x.shape
    half = dim // 2
    hw = h * w
    cdim = dim // _CHUNKS_PER_SLOT

    def body(row_ref, col_ref, out_ref, buf, sem):
        lane = jax.lax.broadcasted_iota(jnp.int32, (w, hw), 1)
        src = jax.lax.broadcasted_iota(jnp.int32, (w, hw), 0)
        p = (lane % w == src).astype(jnp.float32)
        lane_h = jax.lax.broadcasted_iota(jnp.int32, (h, hw), 1)
        src_h = jax.lax.broadcasted_iota(jnp.int32, (h, hw), 0)
        q = (lane_h // w == src_h).astype(jnp.float32)
        xe = jax.lax.dot_general(
            col_ref[...], p, (((0,), (0,)), ((), ())),
            preferred_element_type=jnp.float32,
        )
        ye = jax.lax.dot_general(
            row_ref[...], q, (((0,), (0,)), ((), ())),
            preferred_element_type=jnp.float32,
        )
        buf[0:half, :] = xe
        buf[half:dim, :] = ye
        copies = []
        for k in range(n):
            for c in range(_CHUNKS_PER_SLOT):
                cp = pltpu.make_async_copy(
                    buf.at[pl.ds(c * cdim, cdim)],
                    out_ref.at[k, pl.ds(c * cdim, cdim)],
                    sem.at[k],
                )
                cp.start()
                copies.append(cp)
        for cp in copies:
            cp.wait()

    out = pl.pallas_call(
        body,
        in_specs=[
            pl.BlockSpec(memory_space=pltpu.VMEM),
            pl.BlockSpec(memory_space=pltpu.VMEM),
        ],
        out_specs=pl.BlockSpec(memory_space=pl.ANY),
        out_shape=jax.ShapeDtypeStruct((n, dim, hw), jnp.float32),
        scratch_shapes=[
            pltpu.VMEM((dim, hw), jnp.float32),
            pltpu.SemaphoreType.DMA((n,)),
        ],
    )(row_w[:h], col_w[:w])
    return out.reshape(n, dim, h, w)
